# baseline (speedup 1.0000x reference)
import functools

import jax
import jax.numpy as jnp
from jax import lax
from jax.experimental import pallas as pl
from jax.experimental.pallas import tpu as pltpu
from jax.experimental.pallas import tpu_sc as plsc
from jax.experimental import layout as jax_layout

_BATCH = 16384
_NPOS = 200
_NC, _NS = 2, 16
_NW = _NC * _NS
_CH = 16
_NCHAIN = 4
_CLEN = _NPOS // _NCHAIN

_SC_ROWS = 6144
_RPW = _SC_ROWS // _NW
_NCHUNK = _RPW // _CH

_TC_ROWS = _BATCH - _SC_ROWS
_BB = 2048
_TC_OFF = _SC_ROWS // _BB


def _chunk_argmax(buf_v, emb_v, lane):
    neg_inf = jnp.full((_CH,), -jnp.inf, jnp.float32)
    zero = jnp.zeros((_CH,), jnp.int32)
    chain_base = [jnp.full((_CH,), c * _CLEN, jnp.int32) for c in range(_NCHAIN)]

    def jstep(t, carry):
        tv = jnp.full((_CH,), t, jnp.int32)
        out = []
        for c in range(_NCHAIN):
            cur, idx = carry[2 * c], carry[2 * c + 1]
            jv = chain_base[c] + tv
            v = plsc.load_gather(buf_v, [lane, jv])
            cond = v > cur
            out.append(jnp.where(cond, v, cur))
            out.append(jnp.where(cond, jv, idx))
        return tuple(out)

    init = tuple(x for _ in range(_NCHAIN) for x in (neg_inf, zero))
    carry = lax.fori_loop(0, _CLEN, jstep, init, unroll=5)
    cur, idx = carry[0], carry[1]
    for c in range(1, _NCHAIN):
        cond = carry[2 * c] > cur
        cur = jnp.where(cond, carry[2 * c], cur)
        idx = jnp.where(cond, carry[2 * c + 1], idx)
    return plsc.load_gather(emb_v, [idx])


def _sc_body(pos_hbm, emb_hbm, out_hbm, emb_v, buf0, buf1, out_v, sem0, sem1):
    c = lax.axis_index("c")
    s = lax.axis_index("s")
    wid = s * _NC + c
    base = wid * _RPW
    pltpu.sync_copy(emb_hbm, emb_v)
    lane = lax.iota(jnp.int32, 16)

    def copy_in(chunk, buf, sem):
        return pltpu.async_copy(
            pos_hbm.at[pl.ds(base + chunk * _CH, _CH), :], buf, sem)

    copy_in(0, buf0, sem0)

    def pair(k, _):
        pltpu.make_async_copy(
            pos_hbm.at[pl.ds(base, _CH), :], buf0, sem0).wait()
        copy_in(2 * k + 1, buf1, sem1)
        out_v[pl.ds(2 * k * _CH, _CH)] = _chunk_argmax(buf0, emb_v, lane)
        pltpu.make_async_copy(
            pos_hbm.at[pl.ds(base, _CH), :], buf1, sem1).wait()

        @pl.when(k + 1 < _NCHUNK // 2)
        def _():
            copy_in(2 * k + 2, buf0, sem0)

        out_v[pl.ds((2 * k + 1) * _CH, _CH)] = _chunk_argmax(buf1, emb_v, lane)
        return ()

    lax.fori_loop(0, _NCHUNK // 2, pair, ())
    pltpu.sync_copy(out_v, out_hbm.at[pl.ds(base, _RPW)])


def _sc_argmax_embed(position, embedding):
    return pl.kernel(
        _sc_body,
        out_type=jax.ShapeDtypeStruct((_SC_ROWS,), jnp.float32),
        mesh=plsc.VectorSubcoreMesh(
            core_axis_name="c", subcore_axis_name="s",
            num_cores=_NC, num_subcores=_NS),
        compiler_params=pltpu.CompilerParams(
            use_tc_tiling_on_sc=True, needs_layout_passes=False),
        scratch_types=[
            pltpu.VMEM((_NPOS,), jnp.float32),
            pltpu.VMEM((_CH, _NPOS), jnp.float32),
            pltpu.VMEM((_CH, _NPOS), jnp.float32),
            pltpu.VMEM((_RPW,), jnp.float32),
            pltpu.SemaphoreType.DMA,
            pltpu.SemaphoreType.DMA,
        ],
    )(position, embedding)


def _tc_body(pos_ref, emb_ref, out_ref):
    pos = pos_ref[...]
    m = jnp.max(pos, axis=1, keepdims=True)
    col = lax.broadcasted_iota(jnp.int32, pos.shape, 1)
    cand = jnp.where(pos == m, col, _NPOS)
    idx = jnp.min(cand, axis=1, keepdims=True)
    emb = emb_ref[...]
    val = jnp.max(jnp.where(col == idx, emb, -jnp.inf), axis=1, keepdims=True)
    out_ref[...] = val


def _tc_argmax_embed(position, embedding):
    emb2d = embedding.reshape(1, _NPOS)
    out = pl.pallas_call(
        _tc_body,
        grid=(_TC_ROWS // _BB,),
        in_specs=[
            pl.BlockSpec((_BB, _NPOS), lambda i: (i + _TC_OFF, 0)),
            pl.BlockSpec((1, _NPOS), lambda i: (0, 0)),
        ],
        out_specs=pl.BlockSpec((_BB, 1), lambda i: (i, 0)),
        out_shape=jax.ShapeDtypeStruct((_TC_ROWS, 1), jnp.float32),
    )(position, emb2d)
    return out.reshape(_TC_ROWS)


@jax.jit
def kernel(position, embedding):
    position = jax_layout.with_layout_constraint(
        position, jax_layout.Layout(major_to_minor=(1, 0)))
    sc_out = _sc_argmax_embed(position, embedding)
    tc_out = _tc_argmax_embed(position, embedding)
    return jnp.concatenate([sc_out, tc_out])


# speedup vs baseline: 5.5605x; 5.5605x over previous
import functools

import jax
import jax.numpy as jnp
from jax import lax
from jax.experimental import pallas as pl

_BATCH = 16384
_NPOS = 200
_CB = 4096


def _tc_body(pos_ref, emb_ref, out_ref):
    pos = pos_ref[...]
    m = jnp.max(pos, axis=0, keepdims=True)
    row = lax.broadcasted_iota(jnp.int32, pos.shape, 0)
    cand = jnp.where(pos == m, row, _NPOS)
    idx = jnp.min(cand, axis=0, keepdims=True)
    emb = emb_ref[...]
    val = jnp.max(jnp.where(row == idx, emb, -jnp.inf), axis=0, keepdims=True)
    out_ref[...] = val


@jax.jit
def kernel(position, embedding):
    pos_t = position.T
    emb2d = embedding.reshape(_NPOS, 1)
    out = pl.pallas_call(
        _tc_body,
        grid=(_BATCH // _CB,),
        in_specs=[
            pl.BlockSpec((_NPOS, _CB), lambda i: (0, i)),
            pl.BlockSpec((_NPOS, 1), lambda i: (0, 0)),
        ],
        out_specs=pl.BlockSpec((1, _CB), lambda i: (0, i)),
        out_shape=jax.ShapeDtypeStruct((1, _BATCH), jnp.float32),
    )(pos_t, emb2d)
    return out.reshape(_BATCH)
